# baseline (speedup 1.0000x reference)
import jax
import jax.numpy as jnp
from jax.experimental import pallas as pl
from jax.experimental.pallas import tpu as pltpu

_BLK = 2000
_HALF = _BLK // 2
_P = 64


def _fused_kernel(x_ref,
                  w10_ref, w12_ref, w13_ref, b10_ref, b12_ref, b13_ref, wc1_ref,
                  w20_ref, w22_ref, w23_ref, b20_ref, b22_ref, b23_ref, wc2_ref,
                  linw_ref, linb_ref, out_ref):
    f32 = jnp.float32

    def mm(a, w):
        return jnp.dot(a, w, preferred_element_type=f32)

    x = x_ref[...]
    i = 0.5 * jnp.tanh(mm(x, w10_ref[...]) + b10_ref[...]) + 0.5
    t = jnp.tanh(mm(x, w12_ref[...]) + b12_ref[...])
    c = i * t
    o = 0.5 * jnp.tanh(mm(x, w13_ref[...]) + b13_ref[...]
                       + wc1_ref[...] * c) + 0.5
    h = o * jnp.tanh(c)

    hp = jnp.concatenate([h[:_HALF], h[_HALF:]], axis=1)

    i = 0.5 * jnp.tanh(mm(hp, w20_ref[...]) + b20_ref[...]) + 0.5
    t = jnp.tanh(mm(hp, w22_ref[...]) + b22_ref[...])
    c = i * t
    o = 0.5 * jnp.tanh(mm(hp, w23_ref[...]) + b23_ref[...]
                       + wc2_ref[...] * c) + 0.5
    h2 = o * jnp.tanh(c)

    r = jnp.maximum(h2, 0.0)
    out2 = mm(r, linw_ref[...]) + linb_ref[...]
    out_ref[:_HALF, :] = out2[:, :1]
    out_ref[_HALF:, :] = out2[:, 1:2]


def kernel(x, edge_index, edge_weight, l1_W, l1_b, l1_T0, l1_T1, l1_cb, l1_wc,
           l2_W, l2_b, l2_T0, l2_T1, l2_cb, l2_wc, lin_W, lin_b):
    n, d_in = x.shape

    def padc(a):
        return jnp.pad(a, ((0, 0), (0, _P - a.shape[1])))

    def padr(a):
        return jnp.pad(a, ((0, _P - a.shape[0]), (0, 0)))

    def bdiag(w):
        z = jnp.zeros_like(w)
        return jnp.concatenate([jnp.concatenate([w, z], axis=1),
                                jnp.concatenate([z, w], axis=1)], axis=0)

    def dup(b):
        return jnp.concatenate([b, b], axis=1)

    w10 = padc(0.5 * l1_W[0])
    w12 = padc(l1_W[2])
    w13 = padc(0.5 * l1_W[3])
    b10 = padc(0.5 * (l1_b[0] + l1_cb[0][None, :]))
    b12 = padc(l1_b[2] + l1_cb[2][None, :])
    b13 = padc(0.5 * (l1_b[3] + l1_cb[3][None, :]))
    wc1 = padc(0.5 * l1_wc[2])

    w20 = bdiag(padc(padr(0.5 * l2_W[0])))
    w22 = bdiag(padc(padr(l2_W[2])))
    w23 = bdiag(padc(padr(0.5 * l2_W[3])))
    b20 = dup(padc(0.5 * (l2_b[0] + l2_cb[0][None, :])))
    b22 = dup(padc(l2_b[2] + l2_cb[2][None, :]))
    b23 = dup(padc(0.5 * (l2_b[3] + l2_cb[3][None, :])))
    wc2 = dup(padc(0.5 * l2_wc[2]))

    linw_p = padr(lin_W)
    zcol = jnp.zeros_like(linw_p)
    linw = jnp.concatenate(
        [jnp.concatenate([linw_p, zcol], axis=1),
         jnp.concatenate([zcol, linw_p], axis=1)], axis=0)
    linb = jnp.broadcast_to(lin_b.reshape(1, 1), (1, 2))

    grid = (n // _BLK,)
    full = lambda shape: pl.BlockSpec(shape, lambda *_: (0,) * len(shape))

    return pl.pallas_call(
        _fused_kernel,
        grid=grid,
        in_specs=[
            pl.BlockSpec((_BLK, d_in), lambda i: (i, 0)),
            full((d_in, _P)), full((d_in, _P)), full((d_in, _P)),
            full((1, _P)), full((1, _P)), full((1, _P)), full((1, _P)),
            full((2 * _P, 2 * _P)), full((2 * _P, 2 * _P)),
            full((2 * _P, 2 * _P)),
            full((1, 2 * _P)), full((1, 2 * _P)), full((1, 2 * _P)),
            full((1, 2 * _P)),
            full((2 * _P, 2)), full((1, 2)),
        ],
        out_specs=pl.BlockSpec((_BLK, 1), lambda i: (i, 0)),
        out_shape=jax.ShapeDtypeStruct((n, 1), jnp.float32),
        compiler_params=pltpu.CompilerParams(
            dimension_semantics=("parallel",)),
    )(x,
      w10, w12, w13, b10, b12, b13, wc1,
      w20, w22, w23, b20, b22, b23, wc2,
      linw, linb)


# speedup vs baseline: 1.0018x; 1.0018x over previous
import jax
import jax.numpy as jnp
from jax.experimental import pallas as pl

_BLK = 2000
_HALF = _BLK // 2
_P = 64


def _fused_kernel(x_ref,
                  w10_ref, w12_ref, w13_ref, b10_ref, b12_ref, b13_ref, wc1_ref,
                  w20_ref, w22_ref, w23_ref, b20_ref, b22_ref, b23_ref, wc2_ref,
                  linw_ref, linb_ref, out_ref):
    f32 = jnp.float32

    def mm(a, w):
        return jnp.dot(a, w, preferred_element_type=f32)

    x = x_ref[...]
    i = 0.5 * jnp.tanh(mm(x, w10_ref[...]) + b10_ref[...]) + 0.5
    t = jnp.tanh(mm(x, w12_ref[...]) + b12_ref[...])
    c = i * t
    o = 0.5 * jnp.tanh(mm(x, w13_ref[...]) + b13_ref[...]
                       + wc1_ref[...] * c) + 0.5
    h = o * jnp.tanh(c)

    hp = jnp.concatenate([h[:_HALF], h[_HALF:]], axis=1)

    i = 0.5 * jnp.tanh(mm(hp, w20_ref[...]) + b20_ref[...]) + 0.5
    t = jnp.tanh(mm(hp, w22_ref[...]) + b22_ref[...])
    c = i * t
    o = 0.5 * jnp.tanh(mm(hp, w23_ref[...]) + b23_ref[...]
                       + wc2_ref[...] * c) + 0.5
    h2 = o * jnp.tanh(c)

    r = jnp.maximum(h2, 0.0)
    out2 = mm(r, linw_ref[...]) + linb_ref[...]
    out_ref[:_HALF, :] = out2[:, :1]
    out_ref[_HALF:, :] = out2[:, 1:2]


def kernel(x, edge_index, edge_weight, l1_W, l1_b, l1_T0, l1_T1, l1_cb, l1_wc,
           l2_W, l2_b, l2_T0, l2_T1, l2_cb, l2_wc, lin_W, lin_b):
    n, d_in = x.shape

    def padc(a):
        return jnp.pad(a, ((0, 0), (0, _P - a.shape[1])))

    def padr(a):
        return jnp.pad(a, ((0, _P - a.shape[0]), (0, 0)))

    def bdiag(w):
        z = jnp.zeros_like(w)
        return jnp.concatenate([jnp.concatenate([w, z], axis=1),
                                jnp.concatenate([z, w], axis=1)], axis=0)

    def dup(b):
        return jnp.concatenate([b, b], axis=1)

    w10 = padc(0.5 * l1_W[0])
    w12 = padc(l1_W[2])
    w13 = padc(0.5 * l1_W[3])
    b10 = padc(0.5 * (l1_b[0] + l1_cb[0][None, :]))
    b12 = padc(l1_b[2] + l1_cb[2][None, :])
    b13 = padc(0.5 * (l1_b[3] + l1_cb[3][None, :]))
    wc1 = padc(0.5 * l1_wc[2])

    w20 = bdiag(padc(padr(0.5 * l2_W[0])))
    w22 = bdiag(padc(padr(l2_W[2])))
    w23 = bdiag(padc(padr(0.5 * l2_W[3])))
    b20 = dup(padc(0.5 * (l2_b[0] + l2_cb[0][None, :])))
    b22 = dup(padc(l2_b[2] + l2_cb[2][None, :]))
    b23 = dup(padc(0.5 * (l2_b[3] + l2_cb[3][None, :])))
    wc2 = dup(padc(0.5 * l2_wc[2]))

    linw_p = padr(lin_W)
    zcol = jnp.zeros_like(linw_p)
    linw = jnp.concatenate(
        [jnp.concatenate([linw_p, zcol], axis=1),
         jnp.concatenate([zcol, linw_p], axis=1)], axis=0)
    linb = jnp.broadcast_to(lin_b.reshape(1, 1), (1, 2))

    grid = (n // _BLK,)
    full = lambda shape: pl.BlockSpec(shape, lambda *_: (0,) * len(shape))

    return pl.pallas_call(
        _fused_kernel,
        grid=grid,
        in_specs=[
            pl.BlockSpec((_BLK, d_in), lambda i: (i, 0)),
            full((d_in, _P)), full((d_in, _P)), full((d_in, _P)),
            full((1, _P)), full((1, _P)), full((1, _P)), full((1, _P)),
            full((2 * _P, 2 * _P)), full((2 * _P, 2 * _P)),
            full((2 * _P, 2 * _P)),
            full((1, 2 * _P)), full((1, 2 * _P)), full((1, 2 * _P)),
            full((1, 2 * _P)),
            full((2 * _P, 2)), full((1, 2)),
        ],
        out_specs=pl.BlockSpec((_BLK, 1), lambda i: (i, 0)),
        out_shape=jax.ShapeDtypeStruct((n, 1), jnp.float32),
    )(x,
      w10, w12, w13, b10, b12, b13, wc1,
      w20, w22, w23, b20, b22, b23, wc2,
      linw, linb)


# speedup vs baseline: 1.1044x; 1.1024x over previous
import jax
import jax.numpy as jnp
from jax.experimental import pallas as pl

_BLK = 2000


def _fused_kernel(x_ref,
                  w10_ref, w12_ref, w13_ref, b10_ref, b12_ref, b13_ref, wc1_ref,
                  w20_ref, w22_ref, w23_ref, b20_ref, b22_ref, b23_ref, wc2_ref,
                  linw_ref, linb_ref, out_ref):
    def mm(a, w):
        return jnp.dot(a, w, preferred_element_type=jnp.float32)

    def cell(h, w0, w2, w3, b0, b2, b3, wc):
        i = 0.5 * jnp.tanh(mm(h, w0) + b0) + 0.5
        t = jnp.tanh(mm(h, w2) + b2)
        c = i * t
        o = 0.5 * jnp.tanh(mm(h, w3) + b3 + wc * c) + 0.5
        return o * jnp.tanh(c)

    x = x_ref[...]
    h = cell(x, w10_ref[...], w12_ref[...], w13_ref[...],
             b10_ref[...], b12_ref[...], b13_ref[...], wc1_ref[...])
    h = cell(h, w20_ref[...], w22_ref[...], w23_ref[...],
             b20_ref[...], b22_ref[...], b23_ref[...], wc2_ref[...])
    h = jnp.maximum(h, 0.0)
    out_ref[...] = mm(h, linw_ref[...]) + linb_ref[...]


def kernel(x, edge_index, edge_weight, l1_W, l1_b, l1_T0, l1_T1, l1_cb, l1_wc,
           l2_W, l2_b, l2_T0, l2_T1, l2_cb, l2_wc, lin_W, lin_b):
    n, d_in = x.shape
    d1 = l1_W.shape[2]
    d2 = l2_W.shape[2]

    w10 = 0.5 * l1_W[0]
    w13 = 0.5 * l1_W[3]
    b10 = 0.5 * (l1_b[0] + l1_cb[0][None, :])
    b12 = (l1_b[2] + l1_cb[2][None, :])
    b13 = 0.5 * (l1_b[3] + l1_cb[3][None, :])
    wc1 = 0.5 * l1_wc[2]
    w20 = 0.5 * l2_W[0]
    w23 = 0.5 * l2_W[3]
    b20 = 0.5 * (l2_b[0] + l2_cb[0][None, :])
    b22 = (l2_b[2] + l2_cb[2][None, :])
    b23 = 0.5 * (l2_b[3] + l2_cb[3][None, :])
    wc2 = 0.5 * l2_wc[2]
    linb = lin_b.reshape(1, 1)

    grid = (n // _BLK,)
    full = lambda shape: pl.BlockSpec(shape, lambda i: (0, 0))

    return pl.pallas_call(
        _fused_kernel,
        grid=grid,
        in_specs=[
            pl.BlockSpec((_BLK, d_in), lambda i: (i, 0)),
            full((d_in, d1)), full((d_in, d1)), full((d_in, d1)),
            full((1, d1)), full((1, d1)), full((1, d1)), full((1, d1)),
            full((d1, d2)), full((d1, d2)), full((d1, d2)),
            full((1, d2)), full((1, d2)), full((1, d2)), full((1, d2)),
            full((d2, 1)), full((1, 1)),
        ],
        out_specs=pl.BlockSpec((_BLK, 1), lambda i: (i, 0)),
        out_shape=jax.ShapeDtypeStruct((n, 1), jnp.float32),
    )(x,
      w10, l1_W[2], w13, b10, b12, b13, wc1,
      w20, l2_W[2], w23, b20, b22, b23, wc2,
      lin_W, linb)
